# baseline (speedup 1.0000x reference)
import jax
import jax.numpy as jnp
from jax import lax
from jax.experimental import pallas as pl
from jax.experimental.pallas import tpu as pltpu
from jax.experimental.pallas import tpu_sc as plsc

NC = 2
NS = 16
LANES = 16

K0 = 27
K1 = 17
BLOCK_ROWS = 72
NBUF = 6


def _build_sc_call(B, S, D, scale):
    rows_per_gather = BLOCK_ROWS // NBUF
    idx_per_gather = rows_per_gather * S
    max_k = max(K0, K1)
    stage_len = max_k * BLOCK_ROWS * S
    tail = B % BLOCK_ROWS
    mesh = plsc.VectorSubcoreMesh(
        core_axis_name="c", subcore_axis_name="s", num_cores=NC, num_subcores=NS
    )
    i32 = jnp.int32

    def body(feat_hbm, idx_hbm, out_hbm, idx_v, out_v, *bufs_and_sems):
        rows_bufs = bufs_and_sems[:NBUF]
        sems = bufs_and_sems[NBUF:]
        c = lax.axis_index("c")
        s = lax.axis_index("s")
        n_blocks = lax.select(c == 0, i32(K0), i32(K1))
        row_start = lax.select(
            c == 0,
            s * i32(K0 * BLOCK_ROWS),
            i32(NS * K0 * BLOCK_ROWS) + s * i32(K1 * BLOCK_ROWS),
        )
        pltpu.sync_copy(
            idx_hbm.at[pl.ds(pl.multiple_of(row_start * i32(S), 16), stage_len)],
            idx_v,
        )

        def idx_slice(g):
            off = pl.multiple_of(g * i32(idx_per_gather), 8)
            return idx_v.at[pl.ds(off, idx_per_gather)]

        for b in range(NBUF):
            pltpu.async_copy(
                feat_hbm.at[idx_slice(i32(b))], rows_bufs[b], sems[b]
            )

        def block_body(bi, carry):
            for b in range(NBUF):
                rb = rows_bufs[b]
                pltpu.make_async_copy(feat_hbm.at[idx_slice(i32(0))], rb, sems[b]).wait()

                def row_body(r, inner_carry):
                    base = r * i32(S)
                    orow = i32(b * rows_per_gather) + r
                    for d in range(D // LANES):
                        sl = pl.ds(d * LANES, LANES)
                        vals = [rb[base + i32(j), sl] for j in range(S)]
                        while len(vals) > 1:
                            nxt = [vals[i] + vals[i + 1] for i in range(0, len(vals) - 1, 2)]
                            if len(vals) % 2:
                                nxt.append(vals[-1])
                            vals = nxt
                        out_v[orow, sl] = vals[0] * scale
                    return inner_carry

                lax.fori_loop(i32(0), i32(rows_per_gather), row_body, i32(0))

                @pl.when(bi + i32(1) < n_blocks)
                def _():
                    g = (bi + i32(1)) * i32(NBUF) + i32(b)
                    pltpu.async_copy(feat_hbm.at[idx_slice(g)], rb, sems[b])

            row0 = row_start + bi * i32(BLOCK_ROWS)

            @pl.when(row0 + i32(BLOCK_ROWS) <= i32(B))
            def _():
                pltpu.sync_copy(out_v, out_hbm.at[pl.ds(row0, BLOCK_ROWS)])

            if tail:
                @pl.when((row0 + i32(BLOCK_ROWS) > i32(B)) & (row0 < i32(B)))
                def _():
                    pltpu.sync_copy(
                        out_v.at[pl.ds(0, tail)],
                        out_hbm.at[pl.ds(pl.multiple_of(row0, 8), tail)],
                    )
            return carry

        lax.fori_loop(i32(0), n_blocks, block_body, i32(0))

    return pl.kernel(
        body,
        out_type=jax.ShapeDtypeStruct((B, D), jnp.float32),
        mesh=mesh,
        scratch_types=[
            pltpu.VMEM((stage_len,), jnp.int32),
            pltpu.VMEM((BLOCK_ROWS, D), jnp.float32),
        ]
        + [pltpu.VMEM((idx_per_gather, D), jnp.float32) for _ in range(NBUF)]
        + [pltpu.SemaphoreType.DMA for _ in range(NBUF)],
    )


def kernel(nodes, neigh_indices, num_sample, features):
    del nodes
    B, S = neigh_indices.shape
    N, D = features.shape
    assert D % LANES == 0

    B_pad = NS * BLOCK_ROWS * (K0 + K1)
    assert B_pad >= B, (B_pad, B)
    max_k = max(K0, K1)

    need = (NS * K0 * BLOCK_ROWS + (NS - 1) * K1 * BLOCK_ROWS + max_k * BLOCK_ROWS) * S
    flat_idx = neigh_indices.astype(jnp.int32).reshape(-1)
    pad = max(0, need - flat_idx.shape[0])
    if pad:
        flat_idx = jnp.concatenate([flat_idx, jnp.zeros((pad,), jnp.int32)])

    feats = features.astype(jnp.float32)
    scale = jnp.float32(1.0 / num_sample)

    call = _build_sc_call(B, S, D, scale)
    return call(feats, flat_idx)


# speedup vs baseline: 1.6324x; 1.6324x over previous
import jax
import jax.numpy as jnp
from jax import lax
from jax.experimental import pallas as pl
from jax.experimental.pallas import tpu as pltpu
from jax.experimental.pallas import tpu_sc as plsc

NC = 2
NS = 16
LANES = 16

K0 = 31
K1 = 18
BLOCK_ROWS = 64
NBUF = 8


def _build_sc_call(B, S, D, scale):
    rows_per_gather = BLOCK_ROWS // NBUF
    idx_per_gather = rows_per_gather * S
    max_k = max(K0, K1)
    stage_len = max_k * BLOCK_ROWS * S
    tail = B % BLOCK_ROWS
    mesh = plsc.VectorSubcoreMesh(
        core_axis_name="c", subcore_axis_name="s", num_cores=NC, num_subcores=NS
    )
    i32 = jnp.int32

    def body(feat_hbm, idx_hbm, out_hbm, idx_v, out_v, *bufs_and_sems):
        rows_bufs = bufs_and_sems[:NBUF]
        sems = bufs_and_sems[NBUF:]
        c = lax.axis_index("c")
        s = lax.axis_index("s")
        n_blocks = lax.select(c == 0, i32(K0), i32(K1))
        row_start = lax.select(
            c == 0,
            s * i32(K0 * BLOCK_ROWS),
            i32(NS * K0 * BLOCK_ROWS) + s * i32(K1 * BLOCK_ROWS),
        )
        pltpu.sync_copy(
            idx_hbm.at[pl.ds(pl.multiple_of(row_start * i32(S), 128), stage_len)],
            idx_v,
        )

        def idx_slice(g):
            off = pl.multiple_of(g * i32(idx_per_gather), 16)
            return idx_v.at[pl.ds(off, idx_per_gather)]

        for b in range(NBUF):
            pltpu.async_copy(
                feat_hbm.at[idx_slice(i32(b))], rows_bufs[b], sems[b]
            )

        def block_body(bi, carry):
            for b in range(NBUF):
                rb = rows_bufs[b]
                pltpu.make_async_copy(feat_hbm.at[idx_slice(i32(0))], rb, sems[b]).wait()

                def row_body(r, inner_carry):
                    base = r * i32(S)
                    orow = i32(b * rows_per_gather) + r
                    for d in range(D // LANES):
                        sl = pl.ds(d * LANES, LANES)
                        vals = [rb[base + i32(j), sl] for j in range(S)]
                        while len(vals) > 1:
                            nxt = [vals[i] + vals[i + 1] for i in range(0, len(vals) - 1, 2)]
                            if len(vals) % 2:
                                nxt.append(vals[-1])
                            vals = nxt
                        out_v[orow, sl] = vals[0] * scale
                    return inner_carry

                lax.fori_loop(i32(0), i32(rows_per_gather), row_body, i32(0))

                @pl.when(bi + i32(1) < n_blocks)
                def _():
                    g = (bi + i32(1)) * i32(NBUF) + i32(b)
                    pltpu.async_copy(feat_hbm.at[idx_slice(g)], rb, sems[b])

            row0 = row_start + bi * i32(BLOCK_ROWS)

            @pl.when(row0 + i32(BLOCK_ROWS) <= i32(B))
            def _():
                pltpu.sync_copy(out_v, out_hbm.at[pl.ds(row0, BLOCK_ROWS)])

            if tail:
                @pl.when((row0 + i32(BLOCK_ROWS) > i32(B)) & (row0 < i32(B)))
                def _():
                    pltpu.sync_copy(
                        out_v.at[pl.ds(0, tail)],
                        out_hbm.at[pl.ds(pl.multiple_of(row0, 8), tail)],
                    )
            return carry

        lax.fori_loop(i32(0), n_blocks, block_body, i32(0))

    return pl.kernel(
        body,
        out_type=jax.ShapeDtypeStruct((B, D), jnp.float32),
        mesh=mesh,
        scratch_types=[
            pltpu.VMEM((stage_len,), jnp.int32),
            pltpu.VMEM((BLOCK_ROWS, D), jnp.float32),
        ]
        + [pltpu.VMEM((idx_per_gather, D), jnp.float32) for _ in range(NBUF)]
        + [pltpu.SemaphoreType.DMA for _ in range(NBUF)],
    )


def kernel(nodes, neigh_indices, num_sample, features):
    del nodes
    B, S = neigh_indices.shape
    N, D = features.shape
    assert D % LANES == 0

    B_pad = NS * BLOCK_ROWS * (K0 + K1)
    assert B_pad >= B, (B_pad, B)
    max_k = max(K0, K1)

    need = (NS * K0 * BLOCK_ROWS + (NS - 1) * K1 * BLOCK_ROWS + max_k * BLOCK_ROWS) * S
    flat_idx = neigh_indices.astype(jnp.int32).reshape(-1)
    pad = max(0, need - flat_idx.shape[0])
    if pad:
        flat_idx = jnp.concatenate([flat_idx, jnp.zeros((pad,), jnp.int32)])

    feats = features.astype(jnp.float32)
    scale = jnp.float32(1.0 / num_sample)

    call = _build_sc_call(B, S, D, scale)
    return call(feats, flat_idx)
